# Initial kernel scaffold; baseline (speedup 1.0000x reference)
import functools

import jax
import jax.numpy as jnp
from jax import lax
from jax.experimental import pallas as pl
from jax.experimental.pallas import tpu as pltpu
from jax.experimental.pallas import tpu_sc as plsc

N = 10000
E = 320000
D_IN = 128
D_OUT = 128
H = 64
ITERS = 3
NS = 16
EPT = E // NS
C = 80
RPT = N // NS
RS = 125

_f32 = jnp.float32


def _mm_body(x_ref, w_ref, o_ref):
    o_ref[...] = jnp.dot(x_ref[...], w_ref[...], preferred_element_type=_f32)


def _matmul_split(x, W):
    BM = 2000
    return pl.pallas_call(
        _mm_body,
        grid=(2, N // BM),
        in_specs=[
            pl.BlockSpec((BM, D_IN), lambda c, r: (r, 0)),
            pl.BlockSpec((D_IN, H), lambda c, r: (0, c)),
        ],
        out_specs=pl.BlockSpec((BM, H), lambda c, r: (c * (N // BM) + r, 0)),
        out_shape=jax.ShapeDtypeStruct((2 * N, H), _f32),
    )(x, W)


def _asm_body(lo_ref, hi_ref, b_ref, o_ref):
    o_ref[...] = jnp.concatenate([lo_ref[...], hi_ref[...]], axis=1) + b_ref[...]


def _assemble_bias(fin, b):
    BM = 2000
    return pl.pallas_call(
        _asm_body,
        grid=(N // BM,),
        in_specs=[
            pl.BlockSpec((BM, H), lambda r: (r, 0)),
            pl.BlockSpec((BM, H), lambda r: (N // BM + r, 0)),
            pl.BlockSpec((1, D_OUT), lambda r: (0, 0)),
        ],
        out_specs=pl.BlockSpec((BM, D_OUT), lambda r: (r, 0)),
        out_shape=jax.ShapeDtypeStruct((N, D_OUT), _f32),
    )(fin, fin, b.reshape(1, D_OUT))


def _sc_body(sup_hbm, ei_hbm, ew_hbm, out_hbm, bufa_hbm, bufb_hbm,
             colbuf, idxbuf, rowbuf, wbuf, gbuf, zbuf, acc, sem):
    c = lax.axis_index("c")
    s = lax.axis_index("s")
    row0 = s * RPT
    zero16 = jnp.zeros((16,), _f32)

    def _zinit(i, _):
        zbuf[pl.ds(i * 16, 16)] = zero16
        return 0

    lax.fori_loop(0, (RS * H) // 16, _zinit, 0)

    zblock = zbuf.reshape(RS, H)
    for j in range(RPT // RS):
        pltpu.sync_copy(zblock, acc.at[pl.ds(row0 + j * RS, RS)])

    cn = (c * N).astype(jnp.int32)
    ebase = s * EPT

    def one_iter(src_hbm, dst_hbm):
        plsc.subcore_barrier()

        def chunk_body(g, _):
            off = ebase + g * C
            pltpu.sync_copy(ei_hbm.at[0, pl.ds(off, C)], rowbuf)
            pltpu.sync_copy(ei_hbm.at[1, pl.ds(off, C)], colbuf)
            pltpu.sync_copy(ew_hbm.at[pl.ds(off, C)], wbuf)
            for v in range(C // 16):
                idxbuf[pl.ds(v * 16, 16)] = colbuf[pl.ds(v * 16, 16)] + cn
            pltpu.async_copy(src_hbm.at[idxbuf], gbuf, sem).wait()

            def edge_body(e, _):
                w = wbuf[e]
                for q in range(H // 16):
                    gbuf[e, pl.ds(q * 16, 16)] = gbuf[e, pl.ds(q * 16, 16)] * w
                return 0

            lax.fori_loop(0, C, edge_body, 0)
            pltpu.sync_copy(gbuf, acc.at[rowbuf], add=True)
            return 0

        lax.fori_loop(0, EPT // C, chunk_body, 0)
        plsc.subcore_barrier()
        pltpu.sync_copy(acc.at[pl.ds(row0, RPT)],
                        dst_hbm.at[pl.ds(cn + row0, RPT)])
        for j in range(RPT // RS):
            pltpu.sync_copy(zblock, acc.at[pl.ds(row0 + j * RS, RS)])

    one_iter(sup_hbm, bufa_hbm)
    one_iter(bufa_hbm, bufb_hbm)
    one_iter(bufb_hbm, out_hbm)


@functools.partial(
    pl.kernel,
    out_type=(
        jax.ShapeDtypeStruct((2 * N, H), _f32),
        jax.ShapeDtypeStruct((2 * N, H), _f32),
        jax.ShapeDtypeStruct((2 * N, H), _f32),
    ),
    mesh=plsc.VectorSubcoreMesh(core_axis_name="c", subcore_axis_name="s"),
    scratch_types=[
        pltpu.VMEM((C,), jnp.int32),
        pltpu.VMEM((C,), jnp.int32),
        pltpu.VMEM((C,), jnp.int32),
        pltpu.VMEM((C,), _f32),
        pltpu.VMEM((C, H), _f32),
        pltpu.VMEM((RS * H,), _f32),
        pltpu.VMEM_SHARED((N, H), _f32),
        pltpu.SemaphoreType.DMA,
    ],
)
def _sc_spmm(*refs):
    _sc_body(*refs)


def kernel(x, edge_index, edge_weight, W, b):
    sup = _matmul_split(x, W)
    fin, _, _ = _sc_spmm(sup, edge_index, edge_weight)
    return _assemble_bias(fin, b)


# speedup vs baseline: 4.1390x; 4.1390x over previous
import functools

import jax
import jax.numpy as jnp
from jax import lax
from jax.experimental import pallas as pl
from jax.experimental.pallas import tpu as pltpu
from jax.experimental.pallas import tpu_sc as plsc

N = 10000
NP = 10240
E = 320000
D_IN = 128
D = 128
NS = 16
EPC = E // 2
EPT = EPC // NS
C = 80
RPT = NP // NS
RS = 128

_f32 = jnp.float32


def _mm_body(x_ref, w_ref, o_ref):
    o_ref[...] = jnp.dot(x_ref[...], w_ref[...], preferred_element_type=_f32)


def _matmul(x, W):
    BM = 2000
    return pl.pallas_call(
        _mm_body,
        grid=(N // BM,),
        in_specs=[
            pl.BlockSpec((BM, D_IN), lambda r: (r, 0)),
            pl.BlockSpec((D_IN, D), lambda r: (0, 0)),
        ],
        out_specs=pl.BlockSpec((BM, D), lambda r: (r, 0)),
        out_shape=jax.ShapeDtypeStruct((NP, D), _f32),
    )(x, W)


def _comb_body(p_ref, o_ref):
    o_ref[...] = p_ref[0] + p_ref[1]


def _combine(P):
    BM = 2000
    return pl.pallas_call(
        _comb_body,
        grid=(N // BM,),
        in_specs=[pl.BlockSpec((2, BM, D), lambda r: (0, r, 0))],
        out_specs=pl.BlockSpec((BM, D), lambda r: (r, 0)),
        out_shape=jax.ShapeDtypeStruct((NP, D), _f32),
    )(P)


def _final_body(p_ref, b_ref, o_ref):
    o_ref[...] = p_ref[0] + p_ref[1] + b_ref[...]


def _final(P, b):
    BM = 2000
    return pl.pallas_call(
        _final_body,
        grid=(N // BM,),
        in_specs=[
            pl.BlockSpec((2, BM, D), lambda r: (0, r, 0)),
            pl.BlockSpec((1, D), lambda r: (0, 0)),
        ],
        out_specs=pl.BlockSpec((BM, D), lambda r: (r, 0)),
        out_shape=jax.ShapeDtypeStruct((N, D), _f32),
    )(P, b.reshape(1, D))


def _sc_body(sup_hbm, erow_hbm, ecol_hbm, ew_hbm, p_hbm,
             colbuf, rowbuf, wbuf, gbuf, zbuf, acc, sem):
    c = lax.axis_index("c")
    s = lax.axis_index("s")
    row0 = s * RPT
    zero16 = jnp.zeros((16,), _f32)

    def _zinit(r, _):
        for q in range(D // 16):
            zbuf[r, pl.ds(q * 16, 16)] = zero16
        return 0

    lax.fori_loop(0, RS, _zinit, 0)
    for j in range(RPT // RS):
        pltpu.sync_copy(zbuf, acc.at[pl.ds(row0 + j * RS, RS)])

    ebase = c * EPC + s * EPT
    plsc.subcore_barrier()

    def chunk_body(g, _):
        off = ebase + g * C
        pltpu.sync_copy(erow_hbm.at[pl.ds(off, C)], rowbuf)
        pltpu.sync_copy(ecol_hbm.at[pl.ds(off, C)], colbuf)
        pltpu.sync_copy(ew_hbm.at[pl.ds(off, C)], wbuf)
        pltpu.async_copy(sup_hbm.at[colbuf], gbuf, sem).wait()

        def group_body(j, _):
            base = j * 16
            w16 = wbuf[pl.ds(base, 16)]
            for i in range(16):
                w = w16[i]
                e = base + i
                for q in range(D // 16):
                    gbuf[e, pl.ds(q * 16, 16)] = gbuf[e, pl.ds(q * 16, 16)] * w
            return 0

        lax.fori_loop(0, C // 16, group_body, 0)
        pltpu.sync_copy(gbuf, acc.at[rowbuf], add=True)
        return 0

    lax.fori_loop(0, EPT // C, chunk_body, 0)
    plsc.subcore_barrier()
    pltpu.sync_copy(acc.at[pl.ds(row0, RPT)],
                    p_hbm.at[c, pl.ds(row0, RPT)])


@functools.partial(
    pl.kernel,
    out_type=jax.ShapeDtypeStruct((2, NP, D), _f32),
    mesh=plsc.VectorSubcoreMesh(core_axis_name="c", subcore_axis_name="s"),
    scratch_types=[
        pltpu.VMEM((C,), jnp.int32),
        pltpu.VMEM((C,), jnp.int32),
        pltpu.VMEM((C,), _f32),
        pltpu.VMEM((C, D), _f32),
        pltpu.VMEM((RS, D), _f32),
        pltpu.VMEM_SHARED((NP, D), _f32),
        pltpu.SemaphoreType.DMA,
    ],
)
def _sc_spmm(*refs):
    _sc_body(*refs)


def kernel(x, edge_index, edge_weight, W, b):
    row, col = edge_index[0], edge_index[1]
    sup = _matmul(x, W)
    sup = _combine(_sc_spmm(sup, row, col, edge_weight))
    sup = _combine(_sc_spmm(sup, row, col, edge_weight))
    return _final(_sc_spmm(sup, row, col, edge_weight), b)


# speedup vs baseline: 11.0160x; 2.6615x over previous
import functools

import jax
import jax.numpy as jnp
from jax import lax
from jax.experimental import pallas as pl
from jax.experimental.pallas import tpu as pltpu
from jax.experimental.pallas import tpu_sc as plsc

N = 10000
NP = 10240
E = 320000
D_IN = 128
D = 128
NS = 16
EPC = E // 2
EPT = EPC // NS
C = 80
CH = EPT // C
RPT = NP // NS
RS = 32

_f32 = jnp.float32


def _mm_body(x_ref, w_ref, o_ref):
    o_ref[...] = jnp.dot(x_ref[...], w_ref[...], preferred_element_type=_f32)


def _matmul(x, W):
    BM = 2000
    return pl.pallas_call(
        _mm_body,
        grid=(N // BM,),
        in_specs=[
            pl.BlockSpec((BM, D_IN), lambda r: (r, 0)),
            pl.BlockSpec((D_IN, D), lambda r: (0, 0)),
        ],
        out_specs=pl.BlockSpec((BM, D), lambda r: (r, 0)),
        out_shape=jax.ShapeDtypeStruct((NP, D), _f32),
    )(x, W)


def _comb_body(p_ref, o_ref):
    o_ref[...] = p_ref[0] + p_ref[1]


def _combine(P):
    BM = 2000
    return pl.pallas_call(
        _comb_body,
        grid=(N // BM,),
        in_specs=[pl.BlockSpec((2, BM, D), lambda r: (0, r, 0))],
        out_specs=pl.BlockSpec((BM, D), lambda r: (r, 0)),
        out_shape=jax.ShapeDtypeStruct((NP, D), _f32),
    )(P)


def _final_body(p_ref, b_ref, o_ref):
    o_ref[...] = p_ref[0] + p_ref[1] + b_ref[...]


def _final(P, b):
    BM = 2000
    return pl.pallas_call(
        _final_body,
        grid=(N // BM,),
        in_specs=[
            pl.BlockSpec((2, BM, D), lambda r: (0, r, 0)),
            pl.BlockSpec((1, D), lambda r: (0, 0)),
        ],
        out_specs=pl.BlockSpec((BM, D), lambda r: (r, 0)),
        out_shape=jax.ShapeDtypeStruct((N, D), _f32),
    )(P, b.reshape(1, D))


def _sc_body(sup_hbm, erow_hbm, ecol_hbm, ew_hbm, p_hbm,
             colm, rowbuf0, rowbuf1, wbuf0, wbuf1, gbuf0, gbuf1, zbuf, acc,
             sem0, sem1, seme0, seme1):
    c = lax.axis_index("c")
    s = lax.axis_index("s")
    row0 = s * RPT
    zero16 = jnp.zeros((16,), _f32)

    def _zinit(r, _):
        for q in range(D // 16):
            zbuf[r, pl.ds(q * 16, 16)] = zero16
        return 0

    lax.fori_loop(0, RS, _zinit, 0)
    for j in range(RPT // RS):
        pltpu.sync_copy(zbuf, acc.at[pl.ds(row0 + j * RS, RS)])

    t = c * NS + s
    ebase = t * EPT
    pltpu.sync_copy(ecol_hbm.at[t], colm)
    plsc.subcore_barrier()

    def _gather(g, buf, sem):
        pltpu.async_copy(sup_hbm.at[colm.at[g]], buf, sem)

    def _gdrain(g, buf, sem):
        pltpu.make_async_copy(sup_hbm.at[colm.at[g]], buf, sem).wait()

    def _edges(g, rb, wb, sem):
        off = ebase + g * C
        pltpu.async_copy(erow_hbm.at[pl.ds(off, C)], rb, sem)
        pltpu.async_copy(ew_hbm.at[pl.ds(off, C)], wb, sem)

    def _edrain(g, rb, wb, sem):
        off = ebase + g * C
        pltpu.make_async_copy(erow_hbm.at[pl.ds(off, C)], rb, sem).wait()
        pltpu.make_async_copy(ew_hbm.at[pl.ds(off, C)], wb, sem).wait()

    def _compute(buf, wb):
        def grp(j, _):
            base = j * 16
            w16 = wb[pl.ds(base, 16)]
            for i in range(16):
                w = w16[i]
                e = base + i
                for q in range(D // 16):
                    buf[e, pl.ds(q * 16, 16)] = buf[e, pl.ds(q * 16, 16)] * w
            return 0

        lax.fori_loop(0, C // 16, grp, 0)

    def _scatter(buf, rb):
        pltpu.sync_copy(buf, acc.at[rb], add=True)

    _edges(0, rowbuf0, wbuf0, seme0)
    _gather(0, gbuf0, sem0)

    def pair_body(p, _):
        g0 = p * 2
        _edges(g0 + 1, rowbuf1, wbuf1, seme1)
        _gather(g0 + 1, gbuf1, sem1)
        _gdrain(g0, gbuf0, sem0)
        _edrain(g0, rowbuf0, wbuf0, seme0)
        _compute(gbuf0, wbuf0)
        _scatter(gbuf0, rowbuf0)
        _edges(g0 + 2, rowbuf0, wbuf0, seme0)
        _gather(g0 + 2, gbuf0, sem0)
        _gdrain(g0 + 1, gbuf1, sem1)
        _edrain(g0 + 1, rowbuf1, wbuf1, seme1)
        _compute(gbuf1, wbuf1)
        _scatter(gbuf1, rowbuf1)
        return 0

    lax.fori_loop(0, (CH - 1) // 2, pair_body, 0)
    gl = CH - 1
    _gdrain(gl, gbuf0, sem0)
    _edrain(gl, rowbuf0, wbuf0, seme0)
    _compute(gbuf0, wbuf0)
    _scatter(gbuf0, rowbuf0)

    plsc.subcore_barrier()
    pltpu.sync_copy(acc.at[pl.ds(row0, RPT)],
                    p_hbm.at[c, pl.ds(row0, RPT)])


@functools.partial(
    pl.kernel,
    out_type=jax.ShapeDtypeStruct((2, NP, D), _f32),
    mesh=plsc.VectorSubcoreMesh(core_axis_name="c", subcore_axis_name="s"),
    scratch_types=[
        pltpu.VMEM((CH, C), jnp.int32),
        pltpu.VMEM((C,), jnp.int32),
        pltpu.VMEM((C,), jnp.int32),
        pltpu.VMEM((C,), _f32),
        pltpu.VMEM((C,), _f32),
        pltpu.VMEM((C, D), _f32),
        pltpu.VMEM((C, D), _f32),
        pltpu.VMEM((RS, D), _f32),
        pltpu.VMEM_SHARED((NP, D), _f32),
        pltpu.SemaphoreType.DMA,
        pltpu.SemaphoreType.DMA,
        pltpu.SemaphoreType.DMA,
        pltpu.SemaphoreType.DMA,
    ],
)
def _sc_spmm(*refs):
    _sc_body(*refs)


def kernel(x, edge_index, edge_weight, W, b):
    row = edge_index[0]
    col3 = edge_index[1].reshape(2 * NS, CH, C)
    sup = _matmul(x, W)
    sup = _combine(_sc_spmm(sup, row, col3, edge_weight))
    sup = _combine(_sc_spmm(sup, row, col3, edge_weight))
    return _final(_sc_spmm(sup, row, col3, edge_weight), b)
